# baseline (speedup 1.0000x reference)
import functools

import jax
import jax.numpy as jnp
from jax import lax
from jax.experimental import pallas as pl
from jax.experimental.pallas import tpu as pltpu
from jax.experimental.pallas import tpu_sc as plsc

TOKENS = 32768
D_MODEL = 768
NUM_EXPERTS = 64
BLOCK = 4096

_NC = 2
_NS = 16
_NW = _NC * _NS
_ROWS_W = TOKENS // _NW


def _gate_body(x_ref, w_ref, bt_ref, o_ref):
    scores_t = lax.dot_general(
        w_ref[...],
        x_ref[...],
        (((1,), (1,)), ((), ())),
        preferred_element_type=jnp.float32,
    )
    scores_t = scores_t + bt_ref[...]
    m = jnp.max(scores_t, axis=0, keepdims=True)
    row = lax.broadcasted_iota(jnp.int32, scores_t.shape, 0)
    idx = jnp.min(jnp.where(scores_t == m, row, NUM_EXPERTS), axis=0, keepdims=True)
    oh_t = (row == idx).astype(jnp.float32)
    e1 = lax.broadcasted_iota(jnp.int32, (NUM_EXPERTS, NUM_EXPERTS), 0)
    e2 = lax.broadcasted_iota(jnp.int32, (NUM_EXPERTS, NUM_EXPERTS), 1)
    eye = (e1 == e2).astype(jnp.float32)
    o_ref[...] = lax.dot_general(
        oh_t, eye, (((0,), (0,)), ((), ())), preferred_element_type=jnp.float32
    )


def _gate_one_hot(x, W, bt):
    return pl.pallas_call(
        _gate_body,
        grid=(TOKENS // BLOCK,),
        in_specs=[
            pl.BlockSpec((BLOCK, D_MODEL), lambda i: (i, 0)),
            pl.BlockSpec((NUM_EXPERTS, D_MODEL), lambda i: (0, 0)),
            pl.BlockSpec((NUM_EXPERTS, 1), lambda i: (0, 0)),
        ],
        out_specs=pl.BlockSpec((BLOCK, NUM_EXPERTS), lambda i: (i, 0)),
        out_shape=jax.ShapeDtypeStruct((TOKENS, NUM_EXPERTS), jnp.float32),
    )(x, W, bt)


def _idx_body(x_ref, w_ref, bt_ref, o_ref):
    scores_t = lax.dot_general(
        w_ref[...],
        x_ref[...],
        (((1,), (1,)), ((), ())),
        preferred_element_type=jnp.float32,
    )
    scores_t = scores_t + bt_ref[...]
    m = jnp.max(scores_t, axis=0, keepdims=True)
    row = lax.broadcasted_iota(jnp.int32, scores_t.shape, 0)
    o_ref[...] = jnp.min(jnp.where(scores_t == m, row, NUM_EXPERTS), axis=0)


def _top1_indices(x, W, bt):
    return pl.pallas_call(
        _idx_body,
        grid=(TOKENS // BLOCK,),
        in_specs=[
            pl.BlockSpec((BLOCK, D_MODEL), lambda i: (i, 0)),
            pl.BlockSpec((NUM_EXPERTS, D_MODEL), lambda i: (0, 0)),
            pl.BlockSpec((NUM_EXPERTS, 1), lambda i: (0, 0)),
        ],
        out_specs=pl.BlockSpec((BLOCK,), lambda i: (i,)),
        out_shape=jax.ShapeDtypeStruct((TOKENS,), jnp.int32),
    )(x, W, bt)


@functools.cache
def _sc_one_hot_kernel():
    return functools.partial(
        pl.kernel,
        mesh=plsc.VectorSubcoreMesh(core_axis_name="c", subcore_axis_name="s"),
        out_type=jax.ShapeDtypeStruct((TOKENS * NUM_EXPERTS,), jnp.float32),
        scratch_types=[
            pltpu.VMEM((_ROWS_W,), jnp.int32),
            pltpu.VMEM((_ROWS_W * NUM_EXPERTS,), jnp.float32),
        ],
        compiler_params=pltpu.CompilerParams(needs_layout_passes=False),
    )(_sc_one_hot_body)


def _sc_one_hot_body(idx_hbm, out_hbm, idx_v, buf_v):
    wid = lax.axis_index("s") * _NC + lax.axis_index("c")
    base = wid * _ROWS_W
    pltpu.sync_copy(idx_hbm.at[pl.ds(base, _ROWS_W)], idx_v)

    zero16 = jnp.zeros((16,), jnp.float32)

    def zbody(i, c):
        buf_v[pl.ds(i * 16, 16)] = zero16
        return c

    lax.fori_loop(0, _ROWS_W * NUM_EXPERTS // 16, zbody, 0, unroll=8)

    ones16 = jnp.ones((16,), jnp.float32)
    lane = lax.iota(jnp.int32, 16)

    def sbody(g, c):
        rows = lane + g * 16
        cols = idx_v[pl.ds(g * 16, 16)]
        plsc.store_scatter(buf_v, [rows * NUM_EXPERTS + cols], ones16)
        return c

    lax.fori_loop(0, _ROWS_W // 16, sbody, 0, unroll=4)

    pltpu.sync_copy(
        buf_v, out_hbm.at[pl.ds(base * NUM_EXPERTS, _ROWS_W * NUM_EXPERTS)]
    )


_NCHUNK = 4
_CHUNK = TOKENS // _NCHUNK
_CROWS_W = _CHUNK // _NW


def _chunk_indices(x, W, bt, c):
    nb = _CHUNK // BLOCK
    return pl.pallas_call(
        _idx_body,
        grid=(nb,),
        in_specs=[
            pl.BlockSpec((BLOCK, D_MODEL), lambda i, c=c: (c * nb + i, 0)),
            pl.BlockSpec((NUM_EXPERTS, D_MODEL), lambda i: (0, 0)),
            pl.BlockSpec((NUM_EXPERTS, 1), lambda i: (0, 0)),
        ],
        out_specs=pl.BlockSpec((BLOCK,), lambda i: (i,)),
        out_shape=jax.ShapeDtypeStruct((_CHUNK,), jnp.int32),
    )(x, W, bt)


@functools.cache
def _sc_chunk_kernel():
    return functools.partial(
        pl.kernel,
        mesh=plsc.VectorSubcoreMesh(core_axis_name="c", subcore_axis_name="s"),
        out_type=jax.ShapeDtypeStruct((_CHUNK * NUM_EXPERTS,), jnp.float32),
        scratch_types=[
            pltpu.VMEM((_CROWS_W,), jnp.int32),
            pltpu.VMEM((_CROWS_W * NUM_EXPERTS,), jnp.float32),
        ],
        compiler_params=pltpu.CompilerParams(needs_layout_passes=False),
    )(_sc_chunk_body)


def _sc_chunk_body(idx_hbm, out_hbm, idx_v, buf_v):
    wid = lax.axis_index("s") * _NC + lax.axis_index("c")
    base = wid * _CROWS_W
    pltpu.sync_copy(idx_hbm.at[pl.ds(base, _CROWS_W)], idx_v)

    zero16 = jnp.zeros((16,), jnp.float32)

    def zbody(i, c):
        buf_v[pl.ds(i * 16, 16)] = zero16
        return c

    lax.fori_loop(0, _CROWS_W * NUM_EXPERTS // 16, zbody, 0, unroll=8)

    ones16 = jnp.ones((16,), jnp.float32)
    lane = lax.iota(jnp.int32, 16)

    def sbody(g, c):
        rows = lane + g * 16
        cols = idx_v[pl.ds(g * 16, 16)]
        plsc.store_scatter(buf_v, [rows * NUM_EXPERTS + cols], ones16)
        return c

    lax.fori_loop(0, _CROWS_W // 16, sbody, 0, unroll=4)

    pltpu.sync_copy(
        buf_v, out_hbm.at[pl.ds(base * NUM_EXPERTS, _CROWS_W * NUM_EXPERTS)]
    )


def kernel(x, W, b):
    bt = b.reshape(NUM_EXPERTS, 1)
    idxs = [_chunk_indices(x, W, bt, c) for c in range(_NCHUNK)]
    ohs = [_sc_chunk_kernel()(idxs[c]) for c in range(_NCHUNK)]
    return jnp.concatenate(ohs).reshape(TOKENS, NUM_EXPERTS)


# speedup vs baseline: 2.0276x; 2.0276x over previous
import jax
import jax.numpy as jnp
from jax import lax
from jax.experimental import pallas as pl

TOKENS = 32768
D_MODEL = 768
NUM_EXPERTS = 64
BLOCK = 4096


def _gate_body(x_ref, w_ref, bt_ref, o_ref):
    scores_t = lax.dot_general(
        w_ref[...],
        x_ref[...],
        (((1,), (1,)), ((), ())),
        preferred_element_type=jnp.float32,
    )
    scores_t = scores_t + bt_ref[...]
    m = jnp.max(scores_t, axis=0, keepdims=True)
    row = lax.broadcasted_iota(jnp.int32, scores_t.shape, 0)
    idx = jnp.min(jnp.where(scores_t == m, row, NUM_EXPERTS), axis=0, keepdims=True)
    oh_t = (row == idx).astype(jnp.float32)
    e1 = lax.broadcasted_iota(jnp.int32, (NUM_EXPERTS, NUM_EXPERTS), 0)
    e2 = lax.broadcasted_iota(jnp.int32, (NUM_EXPERTS, NUM_EXPERTS), 1)
    eye = (e1 == e2).astype(jnp.float32)
    o_ref[...] = lax.dot_general(
        oh_t, eye, (((0,), (0,)), ((), ())), preferred_element_type=jnp.float32
    )


def kernel(x, W, b):
    bt = b.reshape(NUM_EXPERTS, 1)
    return pl.pallas_call(
        _gate_body,
        grid=(TOKENS // BLOCK,),
        in_specs=[
            pl.BlockSpec((BLOCK, D_MODEL), lambda i: (i, 0)),
            pl.BlockSpec((NUM_EXPERTS, D_MODEL), lambda i: (0, 0)),
            pl.BlockSpec((NUM_EXPERTS, 1), lambda i: (0, 0)),
        ],
        out_specs=pl.BlockSpec((BLOCK, NUM_EXPERTS), lambda i: (i, 0)),
        out_shape=jax.ShapeDtypeStruct((TOKENS, NUM_EXPERTS), jnp.float32),
    )(x, W, bt)


# speedup vs baseline: 2.0351x; 1.0037x over previous
import jax
import jax.numpy as jnp
from jax import lax
from jax.experimental import pallas as pl
from jax.experimental.pallas import tpu as pltpu

TOKENS = 32768
D_MODEL = 768
NUM_EXPERTS = 64
BLOCK = 4096


def _gate_body(x_ref, w_ref, bt_ref, o_ref):
    scores_t = lax.dot_general(
        w_ref[...],
        x_ref[...],
        (((1,), (1,)), ((), ())),
        preferred_element_type=jnp.float32,
    )
    scores_t = scores_t + bt_ref[...]
    m = jnp.max(scores_t, axis=0, keepdims=True)
    row = lax.broadcasted_iota(jnp.int32, scores_t.shape, 0)
    idx = jnp.min(jnp.where(scores_t == m, row, NUM_EXPERTS), axis=0, keepdims=True)
    oh_t = (row == idx).astype(jnp.float32)
    e1 = lax.broadcasted_iota(jnp.int32, (NUM_EXPERTS, NUM_EXPERTS), 0)
    e2 = lax.broadcasted_iota(jnp.int32, (NUM_EXPERTS, NUM_EXPERTS), 1)
    eye = (e1 == e2).astype(jnp.float32)
    o_ref[...] = lax.dot_general(
        oh_t, eye, (((0,), (0,)), ((), ())), preferred_element_type=jnp.float32
    )


def kernel(x, W, b):
    bt = b.reshape(NUM_EXPERTS, 1)
    return pl.pallas_call(
        _gate_body,
        grid=(TOKENS // BLOCK,),
        in_specs=[
            pl.BlockSpec((BLOCK, D_MODEL), lambda i: (i, 0)),
            pl.BlockSpec((NUM_EXPERTS, D_MODEL), lambda i: (0, 0)),
            pl.BlockSpec((NUM_EXPERTS, 1), lambda i: (0, 0)),
        ],
        out_specs=pl.BlockSpec((BLOCK, NUM_EXPERTS), lambda i: (i, 0)),
        out_shape=jax.ShapeDtypeStruct((TOKENS, NUM_EXPERTS), jnp.float32),
        compiler_params=pltpu.CompilerParams(dimension_semantics=("parallel",)),
    )(x, W, bt)
